# baseline (speedup 1.0000x reference)
import functools
import math

import jax
import jax.numpy as jnp
from jax import lax
from jax.experimental import pallas as pl
from jax.experimental.pallas import tpu as pltpu
from jax.experimental.pallas import tpu_sc as plsc

E = 64
B = 16384
MAX_LAT, MIN_LAT = 57.739133, 54.56094
MAX_LON, MIN_LON = 15.14406, 8.08042

NC, NS, L = 2, 16, 16
NW = NC * NS
ROWS_PER_W = B // NW
NB = 256
NBLK = ROWS_PER_W // NB
GCH = 128
TTAB = 384


def _time_table_body(tw_ref, tb_ref, out_ref):
    i = lax.broadcasted_iota(jnp.int32, (TTAB, E), 0)
    m = (i // 32).astype(jnp.float32)
    d = jnp.minimum(i % 32 + 1, 31).astype(jnp.float32)
    two_pi = 2.0 * math.pi
    ms = jnp.sin(two_pi * (m / 12.0))
    mc = jnp.cos(two_pi * (m / 12.0))
    dsn = jnp.sin(two_pi * (d / 31.0))
    dcs = jnp.cos(two_pi * (d / 31.0))
    w = tw_ref[...]
    out_ref[...] = (ms * w[0:1, :] + mc * w[1:2, :]
                    + dsn * w[2:3, :] + dcs * w[3:4, :] + tb_ref[...])


_time_table = pl.pallas_call(
    _time_table_body,
    out_shape=jax.ShapeDtypeStruct((TTAB, E), jnp.float32),
)


_CMOD_ROWS = 100000
_CBLK = 2000


def _compact_body(in_ref, out_ref):
    x = in_ref[...]
    y = x.reshape(_CBLK // 2, 2, E)
    out_ref[:, 0:E] = y[:, 0, :]
    out_ref[:, E:2 * E] = y[:, 1, :]


_compact = pl.pallas_call(
    _compact_body,
    grid=(_CMOD_ROWS // _CBLK,),
    in_specs=[pl.BlockSpec((_CBLK, E), lambda i: (i, 0))],
    out_specs=pl.BlockSpec((_CBLK // 2, 2 * E), lambda i: (i, 0)),
    out_shape=jax.ShapeDtypeStruct((_CMOD_ROWS // 2, 2 * E), jnp.float32),
)


_sc_mesh = plsc.VectorSubcoreMesh(core_axis_name="c", subcore_axis_name="s")


@functools.partial(
    pl.kernel,
    out_type=jax.ShapeDtypeStruct((B, 6 * E), jnp.float32),
    mesh=_sc_mesh,
    compiler_params=pltpu.CompilerParams(use_tc_tiling_on_sc=False),
    scratch_types=[
        pltpu.VMEM((2, GCH), jnp.int32),
        pltpu.VMEM((2, GCH), jnp.int32),
        pltpu.VMEM((2, GCH), jnp.int32),
        pltpu.VMEM((2, GCH), jnp.int32),
        pltpu.VMEM((2, GCH), jnp.int32),
        pltpu.VMEM((2, GCH), jnp.int32),
        pltpu.VMEM((2, GCH), jnp.int32),
        pltpu.VMEM((NB,), jnp.float32),
        pltpu.VMEM((NB,), jnp.float32),
        pltpu.VMEM((NB, E), jnp.float32),
        pltpu.VMEM((NB, E), jnp.float32),
        pltpu.VMEM((NB, E), jnp.float32),
        pltpu.VMEM((NB, E), jnp.float32),
        pltpu.VMEM((NB, E), jnp.float32),
        pltpu.VMEM((NB, E), jnp.float32),
        pltpu.VMEM((2, E), jnp.float32),
        pltpu.VMEM((E,), jnp.float32),
        pltpu.SemaphoreType.DMA,
        pltpu.SemaphoreType.DMA,
        pltpu.SemaphoreType.DMA,
    ],
)
def _sc_embed(hab_h, sub_h, mon_h, day_h, cmod_h, cmak_h, lat_h, lon_h,
              htab_h, stab_h, ttab_h, ctab_h, ktab_h, gw_h, gb_h, out_h,
              hab_i, sub_i, tidx_i, cmod_i, cmak_i, mon_i, day_i,
              lat_v, lon_v, h_r, s_r, t_r, cm_r, ck_r, g_r, gw_v, gb_v,
              sem_i, sem_g, sem_w):
    wid = lax.axis_index("s") * NC + lax.axis_index("c")
    base_w = wid * ROWS_PER_W

    pltpu.sync_copy(gw_h, gw_v)
    pltpu.sync_copy(gb_h, gb_v)
    g0 = [gw_v[0, pl.ds(c * L, L)] for c in range(E // L)]
    g1 = [gw_v[1, pl.ds(c * L, L)] for c in range(E // L)]
    gb = [gb_v[pl.ds(c * L, L)] for c in range(E // L)]

    for blk in range(NBLK):
        base = base_w + blk * NB

        cps = []
        for j in range(NB // GCH):
            sl = pl.ds(base + j * GCH, GCH)
            cps.append(pltpu.async_copy(hab_h.at[sl], hab_i.at[j], sem_i))
            cps.append(pltpu.async_copy(sub_h.at[sl], sub_i.at[j], sem_i))
            cps.append(pltpu.async_copy(mon_h.at[sl], mon_i.at[j], sem_i))
            cps.append(pltpu.async_copy(day_h.at[sl], day_i.at[j], sem_i))
            cps.append(pltpu.async_copy(cmod_h.at[sl], cmod_i.at[j], sem_i))
            cps.append(pltpu.async_copy(cmak_h.at[sl], cmak_i.at[j], sem_i))
        cps.append(pltpu.async_copy(lat_h.at[pl.ds(base, NB)], lat_v, sem_i))
        cps.append(pltpu.async_copy(lon_h.at[pl.ds(base, NB)], lon_v, sem_i))
        for c in cps:
            c.wait()

        for j in range(NB // GCH):
            for c in range(GCH // L):
                sl = pl.ds(c * L, L)
                tidx_i[j, sl] = (mon_i[j, sl] * 32
                                 + jnp.maximum(day_i[j, sl], 1) - 1)

        gs = []
        for j in range(NB // GCH):
            dsl = pl.ds(j * GCH, GCH)
            gs.append(pltpu.async_copy(htab_h.at[hab_i.at[j]], h_r.at[dsl], sem_g))
            gs.append(pltpu.async_copy(stab_h.at[sub_i.at[j]], s_r.at[dsl], sem_g))
            gs.append(pltpu.async_copy(ttab_h.at[tidx_i.at[j]], t_r.at[dsl], sem_g))
            gs.append(pltpu.async_copy(ctab_h.at[cmod_i.at[j]], cm_r.at[dsl], sem_g))
            gs.append(pltpu.async_copy(ktab_h.at[cmak_i.at[j]], ck_r.at[dsl], sem_g))

        lat_s = 2.0 / (MAX_LAT - MIN_LAT)
        lon_s = 2.0 / (MAX_LON - MIN_LON)
        for c in range(NB // L):
            sl = pl.ds(c * L, L)
            la = (lat_v[sl] - MIN_LAT) * lat_s - 1.0
            lo = (lon_v[sl] - MIN_LON) * lon_s - 1.0
            lat_v[sl] = jnp.minimum(jnp.maximum(la, -1.0), 1.0)
            lon_v[sl] = jnp.minimum(jnp.maximum(lo, -1.0), 1.0)

        gdn = lax.GatherDimensionNumbers(
            offset_dims=(), collapsed_slice_dims=(0,), start_index_map=(0,))

        def _splat(vec, idxv):
            return lax.gather(vec, idxv[:, None], gdn, slice_sizes=(1,),
                              mode=lax.GatherScatterMode.PROMISE_IN_BOUNDS)

        def geo_group(g, carry):
            lat_c = lat_v[pl.ds(g * L, L)]
            lon_c = lon_v[pl.ds(g * L, L)]
            for r16 in range(L):
                idxv = jnp.full((L,), r16, jnp.int32)
                la = _splat(lat_c, idxv)
                lo = _splat(lon_c, idxv)
                r = g * L + r16
                for c in range(E // L):
                    g_r[r, pl.ds(c * L, L)] = la * g0[c] + lo * g1[c] + gb[c]
            return carry

        lax.fori_loop(0, NB // L, geo_group, 0)

        for g in gs:
            g.wait()

        ws = []
        row_sl = pl.ds(base, NB)
        ws.append(pltpu.async_copy(h_r, out_h.at[row_sl, pl.ds(0 * E, E)], sem_w))
        ws.append(pltpu.async_copy(s_r, out_h.at[row_sl, pl.ds(1 * E, E)], sem_w))
        ws.append(pltpu.async_copy(t_r, out_h.at[row_sl, pl.ds(2 * E, E)], sem_w))
        ws.append(pltpu.async_copy(cm_r, out_h.at[row_sl, pl.ds(3 * E, E)], sem_w))
        ws.append(pltpu.async_copy(ck_r, out_h.at[row_sl, pl.ds(4 * E, E)], sem_w))
        ws.append(pltpu.async_copy(g_r, out_h.at[row_sl, pl.ds(5 * E, E)], sem_w))
        for w in ws:
            w.wait()


def kernel(habitat, substrate, month, day, camera_model, camera_maker,
           latitude, longitude,
           habitat_table, substrate_table, cmod_table, cmak_table,
           time_W, time_b, geo_W, geo_b):
    ttab = _time_table(time_W, time_b.reshape(1, E))
    cmod_lin = _compact(cmod_table).reshape(_CMOD_ROWS, E)
    out = _sc_embed(habitat.astype(jnp.int32), substrate.astype(jnp.int32),
                    month.astype(jnp.int32), day.astype(jnp.int32),
                    camera_model.astype(jnp.int32),
                    camera_maker.astype(jnp.int32),
                    latitude, longitude,
                    habitat_table, substrate_table, ttab,
                    cmod_lin, cmak_table, geo_W, geo_b)
    return out


# speedup vs baseline: 1.1984x; 1.1984x over previous
import functools
import math

import jax
import jax.numpy as jnp
from jax import lax
from jax.experimental import pallas as pl
from jax.experimental.pallas import tpu as pltpu
from jax.experimental.pallas import tpu_sc as plsc

E = 64
B = 16384
MAX_LAT, MIN_LAT = 57.739133, 54.56094
MAX_LON, MIN_LON = 15.14406, 8.08042

NC, NS, L = 2, 16, 16
NW = NC * NS
ROWS_PER_W = B // NW
NB = 128
NBLK = ROWS_PER_W // NB
CMC = 32
TTAB = 384
W128 = 2 * E


def _time_table_body(tw_ref, tb_ref, out_ref):
    i = lax.broadcasted_iota(jnp.int32, (TTAB, E), 0)
    m = (i // 32).astype(jnp.float32)
    d = jnp.minimum(i % 32 + 1, 31).astype(jnp.float32)
    two_pi = 2.0 * math.pi
    ms = jnp.sin(two_pi * (m / 12.0))
    mc = jnp.cos(two_pi * (m / 12.0))
    dsn = jnp.sin(two_pi * (d / 31.0))
    dcs = jnp.cos(two_pi * (d / 31.0))
    w = tw_ref[...]
    tab = (ms * w[0:1, :] + mc * w[1:2, :]
           + dsn * w[2:3, :] + dcs * w[3:4, :] + tb_ref[...])
    out_ref[:, 0:E] = tab
    out_ref[:, E:W128] = jnp.zeros((TTAB, E), jnp.float32)


_time_table = pl.pallas_call(
    _time_table_body,
    out_shape=jax.ShapeDtypeStruct((TTAB, W128), jnp.float32),
)


_sc_mesh = plsc.VectorSubcoreMesh(core_axis_name="c", subcore_axis_name="s")


@functools.partial(
    pl.kernel,
    out_type=jax.ShapeDtypeStruct((B, 6 * E), jnp.float32),
    mesh=_sc_mesh,
    compiler_params=pltpu.CompilerParams(use_tc_tiling_on_sc=True),
    scratch_types=[
        pltpu.VMEM((NB,), jnp.int32),
        pltpu.VMEM((NB,), jnp.int32),
        pltpu.VMEM((NB,), jnp.int32),
        pltpu.VMEM((NB,), jnp.int32),
        pltpu.VMEM((NB,), jnp.int32),
        pltpu.VMEM((NB,), jnp.int32),
        pltpu.VMEM((NB,), jnp.int32),
        pltpu.VMEM((NB,), jnp.float32),
        pltpu.VMEM((NB,), jnp.float32),
        pltpu.VMEM((NB, W128), jnp.float32),
        pltpu.VMEM((NB, W128), jnp.float32),
        pltpu.VMEM((NB, W128), jnp.float32),
        pltpu.VMEM((NB, W128), jnp.float32),
        pltpu.VMEM((CMC, 8, E), jnp.float32),
        pltpu.VMEM((2, E), jnp.float32),
        pltpu.VMEM((E,), jnp.float32),
        pltpu.SemaphoreType.DMA,
        pltpu.SemaphoreType.DMA,
        pltpu.SemaphoreType.DMA,
        pltpu.SemaphoreType.DMA,
    ],
)
def _sc_embed(hab_h, sub_h, mon_h, day_h, cmod_h, cmak_h, lat_h, lon_h,
              htab_h, stab_h, ttab_h, ctab_h, ktab_h, gw_h, gb_h, out_h,
              hab_i, sub_i, tidx_i, cmod_i, cmak_i, mon_i, day_i,
              lat_v, lon_v, hs_r, s_r, tc_r, kg_r, grp_v, gw_v, gb_v,
              sem_i, sem_g, sem_w, sem_c):
    wid = lax.axis_index("s") * NC + lax.axis_index("c")
    base_w = wid * ROWS_PER_W

    pltpu.sync_copy(gw_h, gw_v)
    pltpu.sync_copy(gb_h, gb_v)
    g0 = [gw_v[0, pl.ds(c * L, L)] for c in range(E // L)]
    g1 = [gw_v[1, pl.ds(c * L, L)] for c in range(E // L)]
    gb = [gb_v[pl.ds(c * L, L)] for c in range(E // L)]

    gdn = lax.GatherDimensionNumbers(
        offset_dims=(), collapsed_slice_dims=(0,), start_index_map=(0,))

    def _splat(vec, idxv):
        return lax.gather(vec, idxv[:, None], gdn, slice_sizes=(1,),
                          mode=lax.GatherScatterMode.PROMISE_IN_BOUNDS)

    for blk in range(NBLK):
        base = base_w + blk * NB
        sl = pl.ds(base, NB)

        cps = [
            pltpu.async_copy(hab_h.at[sl], hab_i, sem_i),
            pltpu.async_copy(sub_h.at[sl], sub_i, sem_i),
            pltpu.async_copy(mon_h.at[sl], mon_i, sem_i),
            pltpu.async_copy(day_h.at[sl], day_i, sem_i),
            pltpu.async_copy(cmod_h.at[sl], cmod_i, sem_i),
            pltpu.async_copy(cmak_h.at[sl], cmak_i, sem_i),
            pltpu.async_copy(lat_h.at[sl], lat_v, sem_i),
            pltpu.async_copy(lon_h.at[sl], lon_v, sem_i),
        ]
        for c in cps:
            c.wait()

        for c in range(NB // L):
            csl = pl.ds(c * L, L)
            tidx_i[csl] = (mon_i[csl] * 32
                           + jnp.maximum(day_i[csl], 1) - 1)

        gs = [
            pltpu.async_copy(htab_h.at[hab_i], hs_r, sem_g),
            pltpu.async_copy(stab_h.at[sub_i], s_r, sem_g),
            pltpu.async_copy(ttab_h.at[tidx_i], tc_r, sem_g),
            pltpu.async_copy(ktab_h.at[cmak_i], kg_r, sem_g),
        ]

        lat_s = 2.0 / (MAX_LAT - MIN_LAT)
        lon_s = 2.0 / (MAX_LON - MIN_LON)
        for c in range(NB // L):
            csl = pl.ds(c * L, L)
            la = (lat_v[csl] - MIN_LAT) * lat_s - 1.0
            lo = (lon_v[csl] - MIN_LON) * lon_s - 1.0
            lat_v[csl] = jnp.minimum(jnp.maximum(la, -1.0), 1.0)
            lon_v[csl] = jnp.minimum(jnp.maximum(lo, -1.0), 1.0)

        for sc in range(NB // CMC):
            def cm_fire(c, carry, sc=sc):
                v = cmod_i[pl.ds(sc * CMC + c * L, L)]
                v8 = (v >> 3) << 3
                for lane in range(L):
                    i8 = pl.multiple_of(v8[lane], 8)
                    pltpu.async_copy(ctab_h.at[pl.ds(i8, 8)],
                                     grp_v.at[c * L + lane], sem_c)
                return carry

            lax.fori_loop(0, CMC // L, cm_fire, 0)

            def cm_drain(c, carry):
                for lane in range(L):
                    pltpu.make_async_copy(ctab_h.at[pl.ds(0, 8)],
                                          grp_v.at[c * L + lane], sem_c).wait()
                return carry

            lax.fori_loop(0, CMC // L, cm_drain, 0)

            def cm_pack(c, carry, sc=sc):
                v = cmod_i[pl.ds(sc * CMC + c * L, L)]
                for lane in range(L):
                    sub = v[lane] & 7
                    g = c * L + lane
                    r = sc * CMC + g
                    for cc in range(E // L):
                        tc_r[r, pl.ds(E + cc * L, L)] = (
                            grp_v[g, sub, pl.ds(cc * L, L)])
                return carry

            if sc == 0:
                for g in gs:
                    g.wait()
            lax.fori_loop(0, CMC // L, cm_pack, 0)

        def merge_row(r, carry):
            for cc in range(E // L):
                hs_r[r, pl.ds(E + cc * L, L)] = s_r[r, pl.ds(cc * L, L)]
            return carry

        lax.fori_loop(0, NB, merge_row, 0)

        def geo_group(g, carry):
            lat_c = lat_v[pl.ds(g * L, L)]
            lon_c = lon_v[pl.ds(g * L, L)]
            for r16 in range(L):
                idxv = jnp.full((L,), r16, jnp.int32)
                la = _splat(lat_c, idxv)
                lo = _splat(lon_c, idxv)
                r = g * L + r16
                for c in range(E // L):
                    kg_r[r, pl.ds(E + c * L, L)] = (la * g0[c] + lo * g1[c]
                                                    + gb[c])
            return carry

        lax.fori_loop(0, NB // L, geo_group, 0)

        ws = [
            pltpu.async_copy(hs_r, out_h.at[sl, pl.ds(0 * W128, W128)], sem_w),
            pltpu.async_copy(tc_r, out_h.at[sl, pl.ds(1 * W128, W128)], sem_w),
            pltpu.async_copy(kg_r, out_h.at[sl, pl.ds(2 * W128, W128)], sem_w),
        ]
        for w in ws:
            w.wait()


def kernel(habitat, substrate, month, day, camera_model, camera_maker,
           latitude, longitude,
           habitat_table, substrate_table, cmod_table, cmak_table,
           time_W, time_b, geo_W, geo_b):
    ttab = _time_table(time_W, time_b.reshape(1, E))
    pad = ((0, 0), (0, E))
    out = _sc_embed(habitat.astype(jnp.int32), substrate.astype(jnp.int32),
                    month.astype(jnp.int32), day.astype(jnp.int32),
                    camera_model.astype(jnp.int32),
                    camera_maker.astype(jnp.int32),
                    latitude, longitude,
                    jnp.pad(habitat_table, pad), jnp.pad(substrate_table, pad),
                    ttab, cmod_table, jnp.pad(cmak_table, pad),
                    geo_W, geo_b)
    return out


# speedup vs baseline: 1.4979x; 1.2499x over previous
import functools
import math

import jax
import jax.numpy as jnp
from jax import lax
from jax.experimental import pallas as pl
from jax.experimental.pallas import tpu as pltpu
from jax.experimental.pallas import tpu_sc as plsc

E = 64
B = 16384
MAX_LAT, MIN_LAT = 57.739133, 54.56094
MAX_LON, MIN_LON = 15.14406, 8.08042

NC, NS, L = 2, 16, 16
NW = NC * NS
ROWS_PER_W = B // NW
NB = 128
NBLK = ROWS_PER_W // NB
CMC = 32
TTAB = 384
W128 = 2 * E


def _time_table_body(tw_ref, tb_ref, out_ref):
    i = lax.broadcasted_iota(jnp.int32, (TTAB, E), 0)
    m = (i // 32).astype(jnp.float32)
    d = jnp.minimum(i % 32 + 1, 31).astype(jnp.float32)
    two_pi = 2.0 * math.pi
    ms = jnp.sin(two_pi * (m / 12.0))
    mc = jnp.cos(two_pi * (m / 12.0))
    dsn = jnp.sin(two_pi * (d / 31.0))
    dcs = jnp.cos(two_pi * (d / 31.0))
    w = tw_ref[...]
    tab = (ms * w[0:1, :] + mc * w[1:2, :]
           + dsn * w[2:3, :] + dcs * w[3:4, :] + tb_ref[...])
    out_ref[:, 0:E] = tab
    out_ref[:, E:W128] = jnp.zeros((TTAB, E), jnp.float32)


_time_table = pl.pallas_call(
    _time_table_body,
    out_shape=jax.ShapeDtypeStruct((TTAB, W128), jnp.float32),
)


_sc_mesh = plsc.VectorSubcoreMesh(core_axis_name="c", subcore_axis_name="s")


@functools.partial(
    pl.kernel,
    out_type=jax.ShapeDtypeStruct((B, 6 * E), jnp.float32),
    mesh=_sc_mesh,
    compiler_params=pltpu.CompilerParams(use_tc_tiling_on_sc=True),
    scratch_types=[
        pltpu.VMEM((NB,), jnp.int32),
        pltpu.VMEM((NB,), jnp.int32),
        pltpu.VMEM((NB,), jnp.int32),
        pltpu.VMEM((NB,), jnp.int32),
        pltpu.VMEM((NB,), jnp.int32),
        pltpu.VMEM((NB,), jnp.int32),
        pltpu.VMEM((NB,), jnp.int32),
        pltpu.VMEM((NB,), jnp.float32),
        pltpu.VMEM((NB,), jnp.float32),
        pltpu.VMEM((NB, W128), jnp.float32),
        pltpu.VMEM((NB, W128), jnp.float32),
        pltpu.VMEM((NB, W128), jnp.float32),
        pltpu.VMEM((NB, W128), jnp.float32),
        pltpu.VMEM((NB, W128), jnp.float32),
        pltpu.VMEM((2, E), jnp.float32),
        pltpu.VMEM((E,), jnp.float32),
        pltpu.SemaphoreType.DMA,
        pltpu.SemaphoreType.DMA,
        pltpu.SemaphoreType.DMA,
        pltpu.SemaphoreType.DMA,
    ],
)
def _sc_embed(hab_h, sub_h, mon_h, day_h, cmod_h, cmak_h, lat_h, lon_h,
              htab_h, stab_h, ttab_h, ctab_h, ktab_h, gw_h, gb_h, out_h,
              hab_i, sub_i, tidx_i, cmod_i, cmak_i, mon_i, day_i,
              lat_v, lon_v, hs_r, s_r, tc_r, kg_r, cm_r, gw_v, gb_v,
              sem_i, sem_g, sem_w, sem_c):
    wid = lax.axis_index("s") * NC + lax.axis_index("c")
    base_w = wid * ROWS_PER_W

    pltpu.sync_copy(gw_h, gw_v)
    pltpu.sync_copy(gb_h, gb_v)
    g0 = [gw_v[0, pl.ds(c * L, L)] for c in range(E // L)]
    g1 = [gw_v[1, pl.ds(c * L, L)] for c in range(E // L)]
    gb = [gb_v[pl.ds(c * L, L)] for c in range(E // L)]

    gdn = lax.GatherDimensionNumbers(
        offset_dims=(), collapsed_slice_dims=(0,), start_index_map=(0,))

    def _splat(vec, idxv):
        return lax.gather(vec, idxv[:, None], gdn, slice_sizes=(1,),
                          mode=lax.GatherScatterMode.PROMISE_IN_BOUNDS)

    for blk in range(NBLK):
        base = base_w + blk * NB
        sl = pl.ds(base, NB)

        cps = [
            pltpu.async_copy(hab_h.at[sl], hab_i, sem_i),
            pltpu.async_copy(sub_h.at[sl], sub_i, sem_i),
            pltpu.async_copy(mon_h.at[sl], mon_i, sem_i),
            pltpu.async_copy(day_h.at[sl], day_i, sem_i),
            pltpu.async_copy(cmod_h.at[sl], cmod_i, sem_i),
            pltpu.async_copy(cmak_h.at[sl], cmak_i, sem_i),
            pltpu.async_copy(lat_h.at[sl], lat_v, sem_i),
            pltpu.async_copy(lon_h.at[sl], lon_v, sem_i),
        ]
        for c in cps:
            c.wait()

        for c in range(NB // L):
            csl = pl.ds(c * L, L)
            tidx_i[csl] = (mon_i[csl] * 32
                           + jnp.maximum(day_i[csl], 1) - 1)

        gs = [
            pltpu.async_copy(htab_h.at[hab_i], hs_r, sem_g),
            pltpu.async_copy(stab_h.at[sub_i], s_r, sem_g),
            pltpu.async_copy(ttab_h.at[tidx_i], tc_r, sem_g),
            pltpu.async_copy(ktab_h.at[cmak_i], kg_r, sem_g),
            pltpu.async_copy(ctab_h.at[cmod_i], cm_r, sem_g),
        ]

        lat_s = 2.0 / (MAX_LAT - MIN_LAT)
        lon_s = 2.0 / (MAX_LON - MIN_LON)
        for c in range(NB // L):
            csl = pl.ds(c * L, L)
            la = (lat_v[csl] - MIN_LAT) * lat_s - 1.0
            lo = (lon_v[csl] - MIN_LON) * lon_s - 1.0
            lat_v[csl] = jnp.minimum(jnp.maximum(la, -1.0), 1.0)
            lon_v[csl] = jnp.minimum(jnp.maximum(lo, -1.0), 1.0)

        for g in gs:
            g.wait()

        def merge_row(r, carry):
            for cc in range(E // L):
                hs_r[r, pl.ds(E + cc * L, L)] = s_r[r, pl.ds(cc * L, L)]
                tc_r[r, pl.ds(E + cc * L, L)] = cm_r[r, pl.ds(cc * L, L)]
            return carry

        lax.fori_loop(0, NB, merge_row, 0)

        def geo_group(g, carry):
            lat_c = lat_v[pl.ds(g * L, L)]
            lon_c = lon_v[pl.ds(g * L, L)]
            for r16 in range(L):
                idxv = jnp.full((L,), r16, jnp.int32)
                la = _splat(lat_c, idxv)
                lo = _splat(lon_c, idxv)
                r = g * L + r16
                for c in range(E // L):
                    kg_r[r, pl.ds(E + c * L, L)] = (la * g0[c] + lo * g1[c]
                                                    + gb[c])
            return carry

        lax.fori_loop(0, NB // L, geo_group, 0)

        ws = [
            pltpu.async_copy(hs_r, out_h.at[sl, pl.ds(0 * W128, W128)], sem_w),
            pltpu.async_copy(tc_r, out_h.at[sl, pl.ds(1 * W128, W128)], sem_w),
            pltpu.async_copy(kg_r, out_h.at[sl, pl.ds(2 * W128, W128)], sem_w),
        ]
        for w in ws:
            w.wait()


def kernel(habitat, substrate, month, day, camera_model, camera_maker,
           latitude, longitude,
           habitat_table, substrate_table, cmod_table, cmak_table,
           time_W, time_b, geo_W, geo_b):
    ttab = _time_table(time_W, time_b.reshape(1, E))
    pad = ((0, 0), (0, E))
    out = _sc_embed(habitat.astype(jnp.int32), substrate.astype(jnp.int32),
                    month.astype(jnp.int32), day.astype(jnp.int32),
                    camera_model.astype(jnp.int32),
                    camera_maker.astype(jnp.int32),
                    latitude, longitude,
                    jnp.pad(habitat_table, pad), jnp.pad(substrate_table, pad),
                    ttab, jnp.pad(cmod_table, pad), jnp.pad(cmak_table, pad),
                    geo_W, geo_b)
    return out
